# baseline (speedup 1.0000x reference)
import functools

import jax
import jax.numpy as jnp
from jax import lax
from jax.experimental import pallas as pl
from jax.experimental.pallas import tpu as pltpu
from jax.experimental.pallas import tpu_sc as plsc

_NC = 2
_NS = 16
_L = 16
_NW = _NC * _NS
_B = 16384
_CH = _B // _NW

_H = 10
_PERM = (6, 7, 8, 9, 0, 1, 5, 2, 3, 4)

_mesh = plsc.VectorSubcoreMesh(core_axis_name="c", subcore_axis_name="s")


@functools.partial(
    pl.kernel,
    out_type=jax.ShapeDtypeStruct((_B,), jnp.float32),
    mesh=_mesh,
    scratch_types=[
        pltpu.VMEM((_CH,), jnp.float32),
        pltpu.VMEM((_CH,), jnp.float32),
        pltpu.VMEM((_L,), jnp.float32),
        pltpu.VMEM((_L,), jnp.float32),
    ],
)
def _sc_net(x_hbm, w1_hbm, w2_hbm, out_hbm, x_v, y_v, w1_v, w2_v):
    wid = lax.axis_index("s") * _NC + lax.axis_index("c")
    base = wid * _CH
    pltpu.sync_copy(x_hbm.at[pl.ds(base, _CH)], x_v)
    pltpu.sync_copy(w1_hbm, w1_v)
    pltpu.sync_copy(w2_hbm, w2_v)
    w1 = w1_v[...]
    w2 = w2_v[...]
    c = w2[0] * w1[_PERM[0]]
    for j in range(1, _H):
        c = c + w2[j] * w1[_PERM[j]]
    for j in range(_CH // _L):
        sl = pl.ds(j * _L, _L)
        y_v[sl] = x_v[sl] * c
    pltpu.sync_copy(y_v, out_hbm.at[pl.ds(base, _CH)])


def kernel(x, W1, W2):
    xf = x.reshape(_B)
    w1p = jnp.zeros((_L,), jnp.float32).at[:_H].set(W1[:, 0])
    w2p = jnp.zeros((_L,), jnp.float32).at[:_H].set(W2[0, :])
    y = _sc_net(xf, w1p, w2p)
    return y.reshape(_B, 1)


# speedup vs baseline: 1.0898x; 1.0898x over previous
import functools

import jax
import jax.numpy as jnp
from jax import lax
from jax.experimental import pallas as pl
from jax.experimental.pallas import tpu as pltpu
from jax.experimental.pallas import tpu_sc as plsc

_NC = 1
_NS = 16
_L = 16
_NW = _NC * _NS
_B = 16384
_CH = _B // _NW

_H = 10
_PERM = (6, 7, 8, 9, 0, 1, 5, 2, 3, 4)

_mesh = plsc.VectorSubcoreMesh(
    core_axis_name="c", subcore_axis_name="s", num_cores=_NC, num_subcores=_NS
)


@functools.partial(
    pl.kernel,
    out_type=jax.ShapeDtypeStruct((_B,), jnp.float32),
    mesh=_mesh,
    scratch_types=[
        pltpu.VMEM((_CH,), jnp.float32),
        pltpu.VMEM((_CH,), jnp.float32),
        pltpu.VMEM((_L,), jnp.float32),
        pltpu.VMEM((_L,), jnp.float32),
    ],
)
def _sc_net(x_hbm, w1_hbm, w2_hbm, out_hbm, x_v, y_v, w1_v, w2_v):
    wid = lax.axis_index("s") * _NC + lax.axis_index("c")
    base = wid * _CH
    pltpu.sync_copy(x_hbm.at[pl.ds(base, _CH)], x_v)
    pltpu.sync_copy(w1_hbm, w1_v)
    pltpu.sync_copy(w2_hbm, w2_v)
    w1 = w1_v[...]
    w2 = w2_v[...]
    c = w2[0] * w1[_PERM[0]]
    for j in range(1, _H):
        c = c + w2[j] * w1[_PERM[j]]
    for j in range(_CH // _L):
        sl = pl.ds(j * _L, _L)
        y_v[sl] = x_v[sl] * c
    pltpu.sync_copy(y_v, out_hbm.at[pl.ds(base, _CH)])


def kernel(x, W1, W2):
    xf = x.reshape(_B)
    w1p = jnp.zeros((_L,), jnp.float32).at[:_H].set(W1[:, 0])
    w2p = jnp.zeros((_L,), jnp.float32).at[:_H].set(W2[0, :])
    y = _sc_net(xf, w1p, w2p)
    return y.reshape(_B, 1)


# speedup vs baseline: 1.1565x; 1.0612x over previous
import functools

import jax
import jax.numpy as jnp
from jax import lax
from jax.experimental import pallas as pl
from jax.experimental.pallas import tpu as pltpu
from jax.experimental.pallas import tpu_sc as plsc

_NC = 1
_NS = 16
_L = 16
_NW = _NC * _NS
_B = 16384
_CH = _B // _NW

_H = 10
_PERM = (6, 7, 8, 9, 0, 1, 5, 2, 3, 4)

_mesh = plsc.VectorSubcoreMesh(
    core_axis_name="c", subcore_axis_name="s", num_cores=_NC, num_subcores=_NS
)


@functools.partial(
    pl.kernel,
    out_type=jax.ShapeDtypeStruct((_B,), jnp.float32),
    mesh=_mesh,
    scratch_types=[
        pltpu.VMEM((_CH,), jnp.float32),
        pltpu.VMEM((_CH,), jnp.float32),
        pltpu.VMEM((_L,), jnp.float32),
        pltpu.VMEM((_L,), jnp.float32),
        pltpu.SemaphoreType.DMA,
        pltpu.SemaphoreType.DMA,
        pltpu.SemaphoreType.DMA,
    ],
)
def _sc_net(x_hbm, w1_hbm, w2_hbm, out_hbm, x_v, y_v, w1_v, w2_v, sx, s1, s2):
    wid = lax.axis_index("s") * _NC + lax.axis_index("c")
    base = wid * _CH
    cx = pltpu.async_copy(x_hbm.at[pl.ds(base, _CH)], x_v, sx)
    c1 = pltpu.async_copy(w1_hbm, w1_v.at[pl.ds(0, _H)], s1)
    c2 = pltpu.async_copy(w2_hbm, w2_v.at[pl.ds(0, _H)], s2)
    c1.wait()
    c2.wait()
    w1 = w1_v[...]
    w2 = w2_v[...]
    c = w2[0] * w1[_PERM[0]]
    for j in range(1, _H):
        c = c + w2[j] * w1[_PERM[j]]
    cx.wait()

    @pl.loop(0, _CH // _L)
    def _scale(j):
        sl = pl.ds(j * _L, _L)
        y_v[sl] = x_v[sl] * c

    pltpu.sync_copy(y_v, out_hbm.at[pl.ds(base, _CH)])


def kernel(x, W1, W2):
    y = _sc_net(x.reshape(_B), W1.reshape(_H), W2.reshape(_H))
    return y.reshape(_B, 1)


# speedup vs baseline: 1.1611x; 1.0040x over previous
import functools

import jax
import jax.numpy as jnp
from jax import lax
from jax.experimental import pallas as pl
from jax.experimental.pallas import tpu as pltpu
from jax.experimental.pallas import tpu_sc as plsc

_NC = 1
_NS = 16
_L = 16
_NW = _NC * _NS
_B = 16384
_CH = _B // _NW

_H = 10
_PERM = (6, 7, 8, 9, 0, 1, 5, 2, 3, 4)

_mesh = plsc.VectorSubcoreMesh(
    core_axis_name="c", subcore_axis_name="s", num_cores=_NC, num_subcores=_NS
)


@functools.partial(
    pl.kernel,
    out_type=jax.ShapeDtypeStruct((_B,), jnp.float32),
    mesh=_mesh,
    scratch_types=[
        pltpu.VMEM((_CH,), jnp.float32),
        pltpu.VMEM((_L,), jnp.float32),
        pltpu.VMEM((_L,), jnp.float32),
        pltpu.SemaphoreType.DMA,
        pltpu.SemaphoreType.DMA,
        pltpu.SemaphoreType.DMA,
    ],
)
def _sc_net(x_hbm, w1_hbm, w2_hbm, out_hbm, x_v, w1_v, w2_v, sx, s1, s2):
    wid = lax.axis_index("s") * _NC + lax.axis_index("c")
    base = wid * _CH
    cx = pltpu.async_copy(x_hbm.at[pl.ds(base, _CH)], x_v, sx)
    c1 = pltpu.async_copy(w1_hbm, w1_v.at[pl.ds(0, _H)], s1)
    c2 = pltpu.async_copy(w2_hbm, w2_v.at[pl.ds(0, _H)], s2)
    c1.wait()
    c2.wait()
    w1 = w1_v[...]
    w2 = w2_v[...]
    c = w2[0] * w1[_PERM[0]]
    for j in range(1, _H):
        c = c + w2[j] * w1[_PERM[j]]
    cx.wait()

    @pl.loop(0, _CH // _L, unroll=4)
    def _scale(j):
        sl = pl.ds(j * _L, _L)
        x_v[sl] = x_v[sl] * c

    pltpu.sync_copy(x_v, out_hbm.at[pl.ds(base, _CH)])


def kernel(x, W1, W2):
    y = _sc_net(x.reshape(_B), W1.reshape(_H), W2.reshape(_H))
    return y.reshape(_B, 1)
